# baseline (speedup 1.0000x reference)
import functools

import jax
import jax.numpy as jnp
from jax import lax
from jax.experimental import pallas as pl
from jax.experimental.pallas import tpu as pltpu
from jax.experimental.pallas import tpu_sc as plsc

B = 16384
R = 32
PACK = 4
PR = PACK * R
NC = 2
NS = 16
L = 16
NW = NC * NS
BPW = B // NW
CH = 128
NCH = BPW // CH
GRP = CH // L


def _lane_take(x, idx):
    dnums = lax.GatherDimensionNumbers(
        offset_dims=(), collapsed_slice_dims=(0,), start_index_map=(0,))
    return lax.gather(x, idx[:, None], dnums, (1,),
                      mode=lax.GatherScatterMode.PROMISE_IN_BOUNDS)


def _body(u_hbm, i_hbm, j_hbm, U_hbm, V_hbm, out_hbm,
          idx_u, idx_i, idx_j, pk_u, pk_i, pk_j,
          rows_u, rows_i, rows_j, out_v, sem):
    wid = lax.axis_index("s") * NC + lax.axis_index("c")
    base = wid * BPW

    pltpu.sync_copy(u_hbm.at[pl.ds(base, BPW)], idx_u)
    pltpu.sync_copy(i_hbm.at[pl.ds(base, BPW)], idx_i)
    pltpu.sync_copy(j_hbm.at[pl.ds(base, BPW)], idx_j)

    def pack_block(k, carry):
        for c in range(NCH):
            s = pl.ds(c * CH + k * L, L)
            d = pl.ds(k * L, L)
            pk_u[c, d] = lax.shift_right_logical(idx_u[s], 2)
            pk_i[c, d] = lax.shift_right_logical(idx_i[s], 2)
            pk_j[c, d] = lax.shift_right_logical(idx_j[s], 2)
        return carry

    lax.fori_loop(0, CH // L, pack_block, 0)

    lane = lax.iota(jnp.int32, L)
    rots = [(lane + off) & (L - 1) for off in (8, 4, 2, 1)]
    zero = jnp.zeros((L,), jnp.float32)

    for c in range(NCH):
        cps = [
            pltpu.async_copy(U_hbm.at[pk_u.at[c]], rows_u, sem),
            pltpu.async_copy(V_hbm.at[pk_i.at[c]], rows_i, sem),
            pltpu.async_copy(V_hbm.at[pk_j.at[c]], rows_j, sem),
        ]
        for cp in cps:
            cp.wait()

        def group(g, carry, c=c):
            gb = g * L
            sv_u = (idx_u[pl.ds(c * CH + gb, L)] & 3) * R
            sv_i = (idx_i[pl.ds(c * CH + gb, L)] & 3) * R
            sv_j = (idx_j[pl.ds(c * CH + gb, L)] & 3) * R
            acc = zero
            for t in range(L):
                r = gb + t
                su = sv_u[t]
                si = sv_i[t]
                sj = sv_j[t]
                u0 = rows_u[r, pl.ds(su, L)]
                u1 = rows_u[r, pl.ds(su + L, L)]
                i0 = rows_i[r, pl.ds(si, L)]
                i1 = rows_i[r, pl.ds(si + L, L)]
                j0 = rows_j[r, pl.ds(sj, L)]
                j1 = rows_j[r, pl.ds(sj + L, L)]
                s = u0 * (i0 - j0) + u1 * (i1 - j1)
                for rot in rots:
                    s = s + _lane_take(s, rot)
                acc = jnp.where(lane == t, s, acc)
            out_v[pl.ds(c * CH + gb, L)] = 1.0 / (1.0 + jnp.exp(-acc))
            return carry

        lax.fori_loop(0, GRP, group, 0)

    pltpu.sync_copy(out_v, out_hbm.at[pl.ds(base, BPW)])


@jax.jit
def _run(u, i, j, U4, V4):
    mesh = plsc.VectorSubcoreMesh(core_axis_name="c", subcore_axis_name="s")
    f = functools.partial(
        pl.kernel,
        mesh=mesh,
        out_type=jax.ShapeDtypeStruct((B,), jnp.float32),
        scratch_types=[
            pltpu.VMEM((BPW,), jnp.int32),
            pltpu.VMEM((BPW,), jnp.int32),
            pltpu.VMEM((BPW,), jnp.int32),
            pltpu.VMEM((NCH, CH), jnp.int32),
            pltpu.VMEM((NCH, CH), jnp.int32),
            pltpu.VMEM((NCH, CH), jnp.int32),
            pltpu.VMEM((CH, PR), jnp.float32),
            pltpu.VMEM((CH, PR), jnp.float32),
            pltpu.VMEM((CH, PR), jnp.float32),
            pltpu.VMEM((BPW,), jnp.float32),
            pltpu.SemaphoreType.DMA,
        ],
    )(_body)
    return f(u, i, j, U4, V4)


def kernel(u, i, j, U, V):
    U4 = U.reshape(U.shape[0] // PACK, PR)
    V4 = V.reshape(V.shape[0] // PACK, PR)
    return _run(u.astype(jnp.int32), i.astype(jnp.int32), j.astype(jnp.int32),
                U4, V4)
